# baseline (speedup 1.0000x reference)
import functools

import jax
import jax.numpy as jnp
from jax import lax
from jax.experimental import pallas as pl
from jax.experimental.pallas import tpu as pltpu
from jax.experimental.pallas import tpu_sc as plsc

N = 10000
E = 320000
D = 128
DE = 16
DH = D // 2

NC = 2
NS = 16
NW = NC * NS
EPT = E // NW
CH = 80
CH2 = CH // 2
NCHUNK = EPT // CH
ZR = 624
ZTAIL = N - NS * ZR

_HIMASK = -65536


def _pack_rows(y):
  yb = y.astype(jnp.bfloat16).astype(jnp.float32)
  lo = lax.bitcast_convert_type(yb[..., :DH], jnp.int32)
  hi = lax.bitcast_convert_type(yb[..., DH:], jnp.int32)
  return hi | ((lo >> 16) & 65535)


def _sc_msg_body(h_hbm, ep_hbm, idx_hbm, zero_hbm, out_hbm,
                 shared,
                 idxb0, idxb1, idxb2, idxb3,
                 rows0, rows1, epb0, epb1,
                 gsem0, gsem1, esem0, esem1,
                 isrc0, isrc1, isrc2, isrc3,
                 ssem0, ssem1):
  c = lax.axis_index("c")
  s = lax.axis_index("s")
  wid = s * NC + c
  idxb = (idxb0, idxb1, idxb2, idxb3)
  rows = (rows0, rows1)
  epb = (epb0, epb1)
  gsem = (gsem0, gsem1)
  esem = (esem0, esem1)
  isrc = (isrc0, isrc1, isrc2, isrc3)
  ssem = (ssem0, ssem1)

  base = wid * EPT

  def fetch_idx(chunk, q):
    pltpu.async_copy(idx_hbm.at[wid, chunk], idxb[q], isrc[q])

  def fetch_data(chunk, b, q):
    pltpu.make_async_copy(idx_hbm.at[wid, chunk], idxb[q], isrc[q]).wait()

    @pl.when(chunk >= 2)
    def _drain_scatter():
      pltpu.make_async_copy(rows[b], shared.at[idxb[q].at[1]], ssem[b]).wait()

    pltpu.async_copy(h_hbm.at[idxb[q].at[0]], rows[b], gsem[b])
    pltpu.async_copy(ep_hbm.at[pl.ds(base + chunk * CH, CH)], epb[b], esem[b])

  fetch_idx(0, 0)
  fetch_idx(1, 1)
  fetch_data(0, 0, 0)

  pltpu.sync_copy(zero_hbm.at[pl.ds(s * ZR, ZR)],
                  shared.at[pl.ds(s * ZR, ZR)])

  @pl.when(s == NS - 1)
  def _zero_tail():
    pltpu.sync_copy(zero_hbm.at[pl.ds(NS * ZR, ZTAIL)],
                    shared.at[pl.ds(NS * ZR, ZTAIL)])
  plsc.subcore_barrier()

  def quad(i, carry):
    k = i * 4
    for u in range(4):
      chunk = k + u
      b = u % 2
      q = u

      @pl.when(chunk < NCHUNK)
      def _step():
        @pl.when(chunk + 2 < NCHUNK)
        def _next_idx():
          fetch_idx(chunk + 2, (q + 2) % 4)

        pltpu.make_async_copy(h_hbm.at[idxb[q].at[0]], rows[b], gsem[b]).wait()
        pltpu.make_async_copy(ep_hbm.at[pl.ds(0, CH)], epb[b], esem[b]).wait()

        @pl.when(chunk + 1 < NCHUNK)
        def _next_data():
          fetch_data(chunk + 1, 1 - b, (q + 1) % 4)

        @plsc.parallel_loop(0, CH, unroll=4)
        def edge(e):
          for j in range(DH // 16):
            sl = pl.ds(j * 16, 16)
            sl_hi = pl.ds(DH + j * 16, 16)
            ve = epb[b][e, sl]
            lo = (lax.bitcast_convert_type(ve << 16, jnp.float32)
                  + rows[b][e, sl])
            hi = (lax.bitcast_convert_type(ve & _HIMASK, jnp.float32)
                  + rows[b][e, sl_hi])
            rows[b][e, sl] = jnp.maximum(lo, 0.0)
            rows[b][e, sl_hi] = jnp.maximum(hi, 0.0)
        pltpu.async_copy(rows[b], shared.at[idxb[q].at[1]], ssem[b], add=True)
    return carry

  lax.fori_loop(0, (NCHUNK + 3) // 4, quad, 0)
  pltpu.make_async_copy(rows[0], shared.at[idxb[0].at[1]], ssem[0]).wait()
  pltpu.make_async_copy(rows[1], shared.at[idxb[1].at[1]], ssem[1]).wait()
  plsc.subcore_barrier()
  pltpu.sync_copy(shared.at[pl.ds(s * ZR, ZR)],
                  out_hbm.at[c, pl.ds(s * ZR, ZR)])

  @pl.when(s == NS - 1)
  def _dump_tail():
    pltpu.sync_copy(shared.at[pl.ds(NS * ZR, ZTAIL)],
                    out_hbm.at[c, pl.ds(NS * ZR, ZTAIL)])


_sc_msg = pl.kernel(
    _sc_msg_body,
    out_type=jax.ShapeDtypeStruct((NC, N, D), jnp.float32),
    mesh=plsc.VectorSubcoreMesh(core_axis_name="c", subcore_axis_name="s"),
    scratch_types=[
        pltpu.VMEM_SHARED((N, D), jnp.float32),
        pltpu.VMEM((2, CH), jnp.int32),
        pltpu.VMEM((2, CH), jnp.int32),
        pltpu.VMEM((2, CH), jnp.int32),
        pltpu.VMEM((2, CH), jnp.int32),
        pltpu.VMEM((CH, D), jnp.float32),
        pltpu.VMEM((CH, D), jnp.float32),
        pltpu.VMEM((CH, DH), jnp.int32),
        pltpu.VMEM((CH, DH), jnp.int32),
        pltpu.SemaphoreType.DMA,
        pltpu.SemaphoreType.DMA,
        pltpu.SemaphoreType.DMA,
        pltpu.SemaphoreType.DMA,
        pltpu.SemaphoreType.DMA,
        pltpu.SemaphoreType.DMA,
        pltpu.SemaphoreType.DMA,
        pltpu.SemaphoreType.DMA,
        pltpu.SemaphoreType.DMA,
        pltpu.SemaphoreType.DMA,
    ],
)


def _eproj_body(env_ref, act_ref, we0_ref, we1_ref, we2_ref,
                o0_ref, o1_ref, o2_ref):
  env = env_ref[...]
  act = act_ref[...]
  o0_ref[...] = _pack_rows(
      jnp.dot(env, we0_ref[...], preferred_element_type=jnp.float32))
  o1_ref[...] = _pack_rows(
      jnp.dot(act, we1_ref[...], preferred_element_type=jnp.float32))
  o2_ref[...] = _pack_rows(
      jnp.dot(act, we2_ref[...], preferred_element_type=jnp.float32))


def _eproj(env_attr, act_attr, We0, We1, We2):
  BE = 4000
  return pl.pallas_call(
      _eproj_body,
      grid=(E // BE,),
      in_specs=[
          pl.BlockSpec((BE, DE), lambda i: (i, 0)),
          pl.BlockSpec((BE, DE), lambda i: (i, 0)),
          pl.BlockSpec((DE, D), lambda i: (0, 0)),
          pl.BlockSpec((DE, D), lambda i: (0, 0)),
          pl.BlockSpec((DE, D), lambda i: (0, 0)),
      ],
      out_specs=[pl.BlockSpec((BE, DH), lambda i: (i, 0))] * 3,
      out_shape=[jax.ShapeDtypeStruct((E, DH), jnp.int32)] * 3,
  )(env_attr, act_attr, We0, We1, We2)


def _dense_body(h_ref, p_ref, ws_ref, wn_ref, b_ref, g_ref, beta_ref,
                out_ref):
  agg = p_ref[0] + p_ref[1]
  y = (jnp.dot(h_ref[...], ws_ref[...], preferred_element_type=jnp.float32)
       + jnp.dot(agg, wn_ref[...], preferred_element_type=jnp.float32)
       + b_ref[...])
  mu = jnp.mean(y, axis=-1, keepdims=True)
  var = jnp.mean((y - mu) * (y - mu), axis=-1, keepdims=True)
  y = (y - mu) * lax.rsqrt(var + 1e-5) * g_ref[...] + beta_ref[...]
  out_ref[...] = jnp.maximum(y, 0.0)


def _dense(h, parts, Ws, Wn, b, g, beta):
  BN = 1000
  return pl.pallas_call(
      _dense_body,
      grid=(N // BN,),
      in_specs=[
          pl.BlockSpec((BN, D), lambda i: (i, 0)),
          pl.BlockSpec((NC, BN, D), lambda i: (0, i, 0)),
          pl.BlockSpec((D, D), lambda i: (0, 0)),
          pl.BlockSpec((D, D), lambda i: (0, 0)),
          pl.BlockSpec((1, D), lambda i: (0, 0)),
          pl.BlockSpec((1, D), lambda i: (0, 0)),
          pl.BlockSpec((1, D), lambda i: (0, 0)),
      ],
      out_specs=pl.BlockSpec((BN, D), lambda i: (i, 0)),
      out_shape=jax.ShapeDtypeStruct((N, D), jnp.float32),
  )(h, parts, Ws, Wn, b.reshape(1, D), g.reshape(1, D), beta.reshape(1, D))


def _dense_final_body(h_ref, p_ref, ws_ref, wn_ref, b_ref, out_ref):
  agg = p_ref[0] + p_ref[1]
  out_ref[...] = (
      jnp.dot(h_ref[...], ws_ref[...], preferred_element_type=jnp.float32)
      + jnp.dot(agg, wn_ref[...], preferred_element_type=jnp.float32)
      + b_ref[...])


def _dense_final(h, parts, Ws, Wn, b):
  BN = 1000
  return pl.pallas_call(
      _dense_final_body,
      grid=(N // BN,),
      in_specs=[
          pl.BlockSpec((BN, D), lambda i: (i, 0)),
          pl.BlockSpec((NC, BN, D), lambda i: (0, i, 0)),
          pl.BlockSpec((D, D), lambda i: (0, 0)),
          pl.BlockSpec((D, D), lambda i: (0, 0)),
          pl.BlockSpec((1, D), lambda i: (0, 0)),
      ],
      out_specs=pl.BlockSpec((BN, D), lambda i: (i, 0)),
      out_shape=jax.ShapeDtypeStruct((N, D), jnp.float32),
  )(h, parts, Ws, Wn, b.reshape(1, D))


def kernel(x, edge_index, env_edge_attr, act_edge_attr,
           Ws0, Wn0, We0, b0, Ws1, Wn1, We1, b1, Ws2, Wn2, We2, b2,
           g0, beta0, g1, beta1):
  idx3 = (edge_index.reshape(2, NW, NCHUNK, CH)
          .transpose(1, 2, 0, 3))
  zeros = jnp.zeros((N, D), jnp.float32)

  ep0, ep1, ep2 = _eproj(env_edge_attr, act_edge_attr, We0, We1, We2)

  parts = _sc_msg(x, ep0, idx3, zeros)
  h1 = _dense(x, parts, Ws0, Wn0, b0, g0, beta0)
  parts = _sc_msg(h1, ep1, idx3, zeros)
  h2 = _dense(h1, parts, Ws1, Wn1, b1, g1, beta1)
  parts = _sc_msg(h2, ep2, idx3, zeros)
  return _dense_final(h2, parts, Ws2, Wn2, b2)


# speedup vs baseline: 1.1005x; 1.1005x over previous
import functools

import jax
import jax.numpy as jnp
from jax import lax
from jax.experimental import pallas as pl
from jax.experimental.pallas import tpu as pltpu
from jax.experimental.pallas import tpu_sc as plsc

N = 10000
E = 320000
D = 128
DE = 16

NC = 2
NS = 16
NW = NC * NS
EPT = E // NW
CH = 80
NCHUNK = EPT // CH
ZR = 624
ZTAIL = N - NS * ZR


def _sc_msg_body(h_hbm, ep_hbm, src_hbm, dst_hbm, zero_hbm, out_hbm,
                 shared,
                 srcb0, srcb1, srcb2, srcb3,
                 dstb0, dstb1, dstb2, dstb3,
                 rows0, rows1, epb0, epb1,
                 gsem0, gsem1, esem0, esem1,
                 isrc0, isrc1, isrc2, isrc3,
                 idst0, idst1, idst2, idst3):
  c = lax.axis_index("c")
  s = lax.axis_index("s")
  wid = s * NC + c
  srcb = (srcb0, srcb1, srcb2, srcb3)
  dstb = (dstb0, dstb1, dstb2, dstb3)
  rows = (rows0, rows1)
  epb = (epb0, epb1)
  gsem = (gsem0, gsem1)
  esem = (esem0, esem1)
  isrc = (isrc0, isrc1, isrc2, isrc3)
  idst = (idst0, idst1, idst2, idst3)

  pltpu.sync_copy(zero_hbm.at[pl.ds(s * ZR, ZR)],
                  shared.at[pl.ds(s * ZR, ZR)])

  @pl.when(s == NS - 1)
  def _zero_tail():
    pltpu.sync_copy(zero_hbm.at[pl.ds(NS * ZR, ZTAIL)],
                    shared.at[pl.ds(NS * ZR, ZTAIL)])
  plsc.subcore_barrier()

  base = wid * EPT

  def fetch_idx(chunk, q):
    pltpu.async_copy(src_hbm.at[wid, chunk], srcb[q], isrc[q])
    pltpu.async_copy(dst_hbm.at[wid, chunk], dstb[q], idst[q])

  def fetch_data(chunk, b, q):
    pltpu.make_async_copy(src_hbm.at[wid, chunk], srcb[q], isrc[q]).wait()
    pltpu.async_copy(h_hbm.at[srcb[q]], rows[b], gsem[b])
    pltpu.async_copy(ep_hbm.at[pl.ds(base + chunk * CH, CH)], epb[b], esem[b])

  fetch_idx(0, 0)
  fetch_idx(1, 1)
  fetch_data(0, 0, 0)

  def quad(i, carry):
    k = i * 4
    for u in range(4):
      chunk = k + u
      b = u % 2
      q = u

      @pl.when(chunk < NCHUNK)
      def _step():
        @pl.when(chunk + 2 < NCHUNK)
        def _next_idx():
          fetch_idx(chunk + 2, (q + 2) % 4)

        pltpu.make_async_copy(h_hbm.at[srcb[q]], rows[b], gsem[b]).wait()
        pltpu.make_async_copy(ep_hbm.at[pl.ds(0, CH)], epb[b], esem[b]).wait()

        @pl.when(chunk + 1 < NCHUNK)
        def _next_data():
          fetch_data(chunk + 1, 1 - b, (q + 1) % 4)

        def edge(e, carry2):
          for j in range(D // 16):
            sl = pl.ds(j * 16, 16)
            rows[b][e, sl] = jnp.maximum(rows[b][e, sl] + epb[b][e, sl], 0.0)
          return carry2

        lax.fori_loop(0, CH, edge, 0)
        pltpu.make_async_copy(dst_hbm.at[wid, chunk], dstb[q], idst[q]).wait()
        pltpu.sync_copy(rows[b], shared.at[dstb[q]], add=True)
    return carry

  lax.fori_loop(0, (NCHUNK + 3) // 4, quad, 0)
  plsc.subcore_barrier()
  pltpu.sync_copy(shared.at[pl.ds(s * ZR, ZR)],
                  out_hbm.at[c, pl.ds(s * ZR, ZR)])

  @pl.when(s == NS - 1)
  def _dump_tail():
    pltpu.sync_copy(shared.at[pl.ds(NS * ZR, ZTAIL)],
                    out_hbm.at[c, pl.ds(NS * ZR, ZTAIL)])


_sc_msg = pl.kernel(
    _sc_msg_body,
    out_type=jax.ShapeDtypeStruct((NC, N, D), jnp.float32),
    mesh=plsc.VectorSubcoreMesh(core_axis_name="c", subcore_axis_name="s"),
    scratch_types=[
        pltpu.VMEM_SHARED((N, D), jnp.float32),
        pltpu.VMEM((CH,), jnp.int32),
        pltpu.VMEM((CH,), jnp.int32),
        pltpu.VMEM((CH,), jnp.int32),
        pltpu.VMEM((CH,), jnp.int32),
        pltpu.VMEM((CH,), jnp.int32),
        pltpu.VMEM((CH,), jnp.int32),
        pltpu.VMEM((CH,), jnp.int32),
        pltpu.VMEM((CH,), jnp.int32),
        pltpu.VMEM((CH, D), jnp.float32),
        pltpu.VMEM((CH, D), jnp.float32),
        pltpu.VMEM((CH, D), jnp.float32),
        pltpu.VMEM((CH, D), jnp.float32),
        pltpu.SemaphoreType.DMA,
        pltpu.SemaphoreType.DMA,
        pltpu.SemaphoreType.DMA,
        pltpu.SemaphoreType.DMA,
        pltpu.SemaphoreType.DMA,
        pltpu.SemaphoreType.DMA,
        pltpu.SemaphoreType.DMA,
        pltpu.SemaphoreType.DMA,
        pltpu.SemaphoreType.DMA,
        pltpu.SemaphoreType.DMA,
        pltpu.SemaphoreType.DMA,
        pltpu.SemaphoreType.DMA,
    ],
)


def _eproj_body(attr_ref, we_ref, out_ref):
  out_ref[...] = jnp.dot(attr_ref[...], we_ref[...],
                         preferred_element_type=jnp.float32)


def _eproj(attr, We):
  BE = 4000
  return pl.pallas_call(
      _eproj_body,
      grid=(E // BE,),
      in_specs=[
          pl.BlockSpec((BE, DE), lambda i: (i, 0)),
          pl.BlockSpec((DE, D), lambda i: (0, 0)),
      ],
      out_specs=pl.BlockSpec((BE, D), lambda i: (i, 0)),
      out_shape=jax.ShapeDtypeStruct((E, D), jnp.float32),
  )(attr, We)


def _dense_body(h_ref, p_ref, ws_ref, wn_ref, b_ref, g_ref, beta_ref,
                out_ref):
  agg = p_ref[0] + p_ref[1]
  y = (jnp.dot(h_ref[...], ws_ref[...], preferred_element_type=jnp.float32)
       + jnp.dot(agg, wn_ref[...], preferred_element_type=jnp.float32)
       + b_ref[...])
  mu = jnp.mean(y, axis=-1, keepdims=True)
  var = jnp.mean((y - mu) * (y - mu), axis=-1, keepdims=True)
  y = (y - mu) * lax.rsqrt(var + 1e-5) * g_ref[...] + beta_ref[...]
  out_ref[...] = jnp.maximum(y, 0.0)


def _dense(h, parts, Ws, Wn, b, g, beta):
  BN = 1000
  return pl.pallas_call(
      _dense_body,
      grid=(N // BN,),
      in_specs=[
          pl.BlockSpec((BN, D), lambda i: (i, 0)),
          pl.BlockSpec((NC, BN, D), lambda i: (0, i, 0)),
          pl.BlockSpec((D, D), lambda i: (0, 0)),
          pl.BlockSpec((D, D), lambda i: (0, 0)),
          pl.BlockSpec((1, D), lambda i: (0, 0)),
          pl.BlockSpec((1, D), lambda i: (0, 0)),
          pl.BlockSpec((1, D), lambda i: (0, 0)),
      ],
      out_specs=pl.BlockSpec((BN, D), lambda i: (i, 0)),
      out_shape=jax.ShapeDtypeStruct((N, D), jnp.float32),
  )(h, parts, Ws, Wn, b.reshape(1, D), g.reshape(1, D), beta.reshape(1, D))


def _dense_final_body(h_ref, p_ref, ws_ref, wn_ref, b_ref, out_ref):
  agg = p_ref[0] + p_ref[1]
  out_ref[...] = (
      jnp.dot(h_ref[...], ws_ref[...], preferred_element_type=jnp.float32)
      + jnp.dot(agg, wn_ref[...], preferred_element_type=jnp.float32)
      + b_ref[...])


def _dense_final(h, parts, Ws, Wn, b):
  BN = 1000
  return pl.pallas_call(
      _dense_final_body,
      grid=(N // BN,),
      in_specs=[
          pl.BlockSpec((BN, D), lambda i: (i, 0)),
          pl.BlockSpec((NC, BN, D), lambda i: (0, i, 0)),
          pl.BlockSpec((D, D), lambda i: (0, 0)),
          pl.BlockSpec((D, D), lambda i: (0, 0)),
          pl.BlockSpec((1, D), lambda i: (0, 0)),
      ],
      out_specs=pl.BlockSpec((BN, D), lambda i: (i, 0)),
      out_shape=jax.ShapeDtypeStruct((N, D), jnp.float32),
  )(h, parts, Ws, Wn, b.reshape(1, D))


def kernel(x, edge_index, env_edge_attr, act_edge_attr,
           Ws0, Wn0, We0, b0, Ws1, Wn1, We1, b1, Ws2, Wn2, We2, b2,
           g0, beta0, g1, beta1):
  src3 = edge_index[0].reshape(NW, NCHUNK, CH)
  dst3 = edge_index[1].reshape(NW, NCHUNK, CH)
  zeros = jnp.zeros((N, D), jnp.float32)

  ep0 = _eproj(env_edge_attr, We0)
  ep1 = _eproj(act_edge_attr, We1)
  ep2 = _eproj(act_edge_attr, We2)

  parts = _sc_msg(x, ep0, src3, dst3, zeros)
  h1 = _dense(x, parts, Ws0, Wn0, b0, g0, beta0)
  parts = _sc_msg(h1, ep1, src3, dst3, zeros)
  h2 = _dense(h1, parts, Ws1, Wn1, b1, g1, beta1)
  parts = _sc_msg(h2, ep2, src3, dst3, zeros)
  return _dense_final(h2, parts, Ws2, Wn2, b2)
